# Initial kernel scaffold; baseline (speedup 1.0000x reference)
import jax
import jax.numpy as jnp
from jax.experimental import pallas as pl


def kernel(feat_map, conv1_w, conv1_b, reg_w, reg_b, cls_w, cls_b, anchors):
    raise NotImplementedError("write your pallas kernel here")


# speedup vs baseline: 2.9831x; 2.9831x over previous
import jax
import jax.numpy as jnp
from jax.experimental import pallas as pl
from jax.experimental.pallas import tpu as pltpu

_PRE_NMS = 6000
_POST_NMS = 300
_NMS_THRESH = 0.7
_MIN_SIZE = 16.0
_B, _C, _H, _W = 8, 512, 64, 64
_A = 9
_IMG = 1024.0
_P = 6016
_PR = _P // 128
_TH = 8
_NS = _H // _TH
_M = _TH * _W


def _head_kernel(x_ref, wcol_ref, b1_ref, wr_ref, br_ref, wc_ref, bc_ref,
                 anc_ref, boxes_ref, sc_ref, patch_scr):
    def strip(s, _):
        r0 = s * _TH
        for t in range(9):
            dy, dx = t // 3, t % 3
            xs = x_ref[0, pl.ds(r0 + dy, _TH), dx:dx + 64, :]
            patch_scr[:, t * 512:(t + 1) * 512] = xs.reshape(_M, 512)
        hid = jnp.maximum(
            jnp.dot(patch_scr[...], wcol_ref[...],
                    preferred_element_type=jnp.float32) + b1_ref[...], 0.0)
        locs = jnp.dot(hid, wr_ref[...],
                       preferred_element_type=jnp.float32) + br_ref[...]
        cls = jnp.dot(hid, wc_ref[...],
                      preferred_element_type=jnp.float32) + bc_ref[...]

        c = cls.reshape(_M, _A, 2)
        c0, c1 = c[..., 0], c[..., 1]
        mx = jnp.maximum(c0, c1)
        e0 = jnp.exp(c0 - mx)
        e1 = jnp.exp(c1 - mx)
        fg = (e1 / (e0 + e1)).transpose(1, 0)

        hw0 = r0 * 64
        a4 = anc_ref[:, pl.ds(hw0, _M)].reshape(_A, 4, _M)
        l4 = locs.transpose(1, 0).reshape(_A, 4, _M)
        ay1, ax1, ay2, ax2 = a4[:, 0], a4[:, 1], a4[:, 2], a4[:, 3]
        ah = ay2 - ay1
        aw = ax2 - ax1
        acy = ay1 + 0.5 * ah
        acx = ax1 + 0.5 * aw
        cy = l4[:, 0] * ah + acy
        cx = l4[:, 1] * aw + acx
        bh = jnp.exp(l4[:, 2]) * ah
        bw = jnp.exp(l4[:, 3]) * aw
        y1 = jnp.minimum(jnp.maximum(cy - 0.5 * bh, 0.0), _IMG)
        x1 = jnp.minimum(jnp.maximum(cx - 0.5 * bw, 0.0), _IMG)
        y2 = jnp.minimum(jnp.maximum(cy + 0.5 * bh, 0.0), _IMG)
        x2 = jnp.minimum(jnp.maximum(cx + 0.5 * bw, 0.0), _IMG)
        hh = y2 - y1
        ww = x2 - x1
        sc = jnp.where((hh >= _MIN_SIZE) & (ww >= _MIN_SIZE), fg, -jnp.inf)

        box = jnp.stack([y1, x1, y2, x2], axis=1).reshape(_A * 4, _M)
        boxes_ref[0, :, pl.ds(hw0, _M)] = box
        sc_ref[0, :, pl.ds(hw0, _M)] = sc
        return 0

    jax.lax.fori_loop(0, _NS, strip, 0, unroll=False)


def _nms_kernel(tb_ref, ts_ref, out_ref, s_scr, a2_scr):
    s_scr[...] = ts_ref[0]
    a2_scr[...] = ((tb_ref[0, 2] - tb_ref[0, 0]) *
                   (tb_ref[0, 3] - tb_ref[0, 1]))
    fi = (jax.lax.broadcasted_iota(jnp.int32, (_PR, 128), 0) * 128 +
          jax.lax.broadcasted_iota(jnp.int32, (_PR, 128), 1))
    lane = jax.lax.broadcasted_iota(jnp.int32, (1, 128), 1)
    m0 = (lane == 0).astype(jnp.float32)
    m1 = (lane == 1).astype(jnp.float32)
    m2 = (lane == 2).astype(jnp.float32)
    m3 = (lane == 3).astype(jnp.float32)
    m4 = (lane == 4).astype(jnp.float32)
    ninf = jnp.float32(-jnp.inf)

    def body(i, _):
        s = s_scr[...]
        m = jnp.max(s, axis=(0, 1), keepdims=True)
        ok = m > ninf
        idx = jnp.min(jnp.where(s == m, fi, jnp.int32(2 ** 30)),
                      axis=(0, 1), keepdims=True)
        sel = fi == idx
        by1 = jnp.max(jnp.where(sel, tb_ref[0, 0], ninf), axis=(0, 1), keepdims=True)
        bx1 = jnp.max(jnp.where(sel, tb_ref[0, 1], ninf), axis=(0, 1), keepdims=True)
        by2 = jnp.max(jnp.where(sel, tb_ref[0, 2], ninf), axis=(0, 1), keepdims=True)
        bx2 = jnp.max(jnp.where(sel, tb_ref[0, 3], ninf), axis=(0, 1), keepdims=True)
        yy1 = jnp.maximum(by1, tb_ref[0, 0])
        xx1 = jnp.maximum(bx1, tb_ref[0, 1])
        yy2 = jnp.minimum(by2, tb_ref[0, 2])
        xx2 = jnp.minimum(bx2, tb_ref[0, 3])
        inter = jnp.maximum(yy2 - yy1, 0.0) * jnp.maximum(xx2 - xx1, 0.0)
        a1 = (by2 - by1) * (bx2 - bx1)
        iou = inter / (a1 + a2_scr[...] - inter + 1e-9)
        s_scr[...] = jnp.where(iou <= _NMS_THRESH, s, ninf)
        okf = jnp.where(ok, 1.0, 0.0)
        row = okf * (m0 * by1 + m1 * bx1 + m2 * by2 + m3 * bx2 +
                     m4 * jnp.where(ok, m, 0.0))
        out_ref[0, pl.ds(i, 1), :] = row
        return 0

    jax.lax.fori_loop(0, _POST_NMS, body, 0, unroll=False)


def _run_heads(xpad, wcol, b1, wr, br, wc, bc, anc):
    return pl.pallas_call(
        _head_kernel,
        grid=(_B,),
        in_specs=[
            pl.BlockSpec((1, 66, 66, 512), lambda b: (b, 0, 0, 0)),
            pl.BlockSpec((4608, 512), lambda b: (0, 0)),
            pl.BlockSpec((1, 512), lambda b: (0, 0)),
            pl.BlockSpec((512, 36), lambda b: (0, 0)),
            pl.BlockSpec((1, 36), lambda b: (0, 0)),
            pl.BlockSpec((512, 18), lambda b: (0, 0)),
            pl.BlockSpec((1, 18), lambda b: (0, 0)),
            pl.BlockSpec((36, 4096), lambda b: (0, 0)),
        ],
        out_specs=[
            pl.BlockSpec((1, 36, 4096), lambda b: (b, 0, 0)),
            pl.BlockSpec((1, 9, 4096), lambda b: (b, 0, 0)),
        ],
        out_shape=[
            jax.ShapeDtypeStruct((_B, 36, 4096), jnp.float32),
            jax.ShapeDtypeStruct((_B, 9, 4096), jnp.float32),
        ],
        scratch_shapes=[pltpu.VMEM((_M, 4608), jnp.float32)],
        compiler_params=pltpu.CompilerParams(
            dimension_semantics=("parallel",),
            vmem_limit_bytes=100 * 1024 * 1024,
        ),
        name="rpn_heads",
    )(xpad, wcol, b1, wr, br, wc, bc, anc)


def _run_nms(tb_p, ts_p):
    return pl.pallas_call(
        _nms_kernel,
        grid=(_B,),
        in_specs=[
            pl.BlockSpec((1, 4, _PR, 128), lambda b: (b, 0, 0, 0)),
            pl.BlockSpec((1, _PR, 128), lambda b: (b, 0, 0)),
        ],
        out_specs=pl.BlockSpec((1, _POST_NMS, 128), lambda b: (b, 0, 0)),
        out_shape=jax.ShapeDtypeStruct((_B, _POST_NMS, 128), jnp.float32),
        scratch_shapes=[pltpu.VMEM((_PR, 128), jnp.float32),
                        pltpu.VMEM((_PR, 128), jnp.float32)],
        compiler_params=pltpu.CompilerParams(
            dimension_semantics=("parallel",),
        ),
        name="rpn_nms",
    )(tb_p, ts_p)


def kernel(feat_map, conv1_w, conv1_b, reg_w, reg_b, cls_w, cls_b, anchors):
    xpad = jnp.pad(feat_map.transpose(0, 2, 3, 1),
                   ((0, 0), (1, 1), (1, 1), (0, 0)))
    wcol = conv1_w.transpose(2, 3, 1, 0).reshape(9 * 512, 512)
    b1 = conv1_b.reshape(1, 512)
    wr = reg_w.reshape(_A * 4, 512).T
    br = reg_b.reshape(1, _A * 4)
    wc = cls_w.reshape(_A * 2, 512).T
    bc = cls_b.reshape(1, _A * 2)
    anc = anchors.reshape(4096, _A * 4).T

    boxesp, scp = _run_heads(xpad, wcol, b1, wr, br, wc, bc, anc)
    boxes = boxesp.transpose(0, 2, 1).reshape(_B, _H * _W * _A, 4)
    sc = scp.transpose(0, 2, 1).reshape(_B, _H * _W * _A)

    top_s, top_i = jax.lax.top_k(sc, _PRE_NMS)
    tb = jnp.take_along_axis(boxes, top_i[..., None], axis=1)

    pad_n = _P - _PRE_NMS
    ts_p = jnp.concatenate(
        [top_s, jnp.full((_B, pad_n), -jnp.inf, jnp.float32)], axis=1
    ).reshape(_B, _PR, 128)
    tb_p = jnp.concatenate(
        [tb, jnp.zeros((_B, pad_n, 4), jnp.float32)], axis=1
    ).transpose(0, 2, 1).reshape(_B, 4, _PR, 128)

    out = _run_nms(tb_p, ts_p)
    return out[:, :, :4], out[:, :, 4]


# speedup vs baseline: 3.3366x; 1.1185x over previous
import jax
import jax.numpy as jnp
from jax.experimental import pallas as pl
from jax.experimental.pallas import tpu as pltpu

_PRE_NMS = 6000
_POST_NMS = 300
_NMS_THRESH = 0.7
_MIN_SIZE = 16.0
_B, _C, _H, _W = 8, 512, 64, 64
_A = 9
_IMG = 1024.0
_P = 6016
_PR = _P // 128
_TH = 8
_NS = _H // _TH
_M = _TH * _W


def _head_kernel(x_ref, wcol_ref, b1_ref, wr_ref, br_ref, wc_ref, bc_ref,
                 anc_ref, boxes_ref, sc_ref):
    def strip(s, _):
        r0 = s * _TH
        acc = None
        for t in range(9):
            dy, dx = t // 3, t % 3
            xs = x_ref[0, pl.ds(r0 + dy, _TH), dx:dx + 64, :].reshape(_M, 512)
            p = jnp.dot(xs, wcol_ref[pl.ds(t * 512, 512), :],
                        preferred_element_type=jnp.float32)
            acc = p if acc is None else acc + p
        hid = jnp.maximum(acc + b1_ref[...], 0.0)
        locs = jnp.dot(hid, wr_ref[...],
                       preferred_element_type=jnp.float32) + br_ref[...]
        cls = jnp.dot(hid, wc_ref[...],
                      preferred_element_type=jnp.float32) + bc_ref[...]

        c = cls.reshape(_M, _A, 2)
        c0, c1 = c[..., 0], c[..., 1]
        mx = jnp.maximum(c0, c1)
        e0 = jnp.exp(c0 - mx)
        e1 = jnp.exp(c1 - mx)
        fg = (e1 / (e0 + e1)).transpose(1, 0)

        hw0 = r0 * 64
        a4 = anc_ref[:, pl.ds(hw0, _M)].reshape(_A, 4, _M)
        l4 = locs.transpose(1, 0).reshape(_A, 4, _M)
        ay1, ax1, ay2, ax2 = a4[:, 0], a4[:, 1], a4[:, 2], a4[:, 3]
        ah = ay2 - ay1
        aw = ax2 - ax1
        acy = ay1 + 0.5 * ah
        acx = ax1 + 0.5 * aw
        cy = l4[:, 0] * ah + acy
        cx = l4[:, 1] * aw + acx
        bh = jnp.exp(l4[:, 2]) * ah
        bw = jnp.exp(l4[:, 3]) * aw
        y1 = jnp.minimum(jnp.maximum(cy - 0.5 * bh, 0.0), _IMG)
        x1 = jnp.minimum(jnp.maximum(cx - 0.5 * bw, 0.0), _IMG)
        y2 = jnp.minimum(jnp.maximum(cy + 0.5 * bh, 0.0), _IMG)
        x2 = jnp.minimum(jnp.maximum(cx + 0.5 * bw, 0.0), _IMG)
        hh = y2 - y1
        ww = x2 - x1
        sc = jnp.where((hh >= _MIN_SIZE) & (ww >= _MIN_SIZE), fg, -jnp.inf)

        box = jnp.stack([y1, x1, y2, x2], axis=1).reshape(_A * 4, _M)
        boxes_ref[0, :, pl.ds(hw0, _M)] = box
        sc_ref[0, :, pl.ds(hw0, _M)] = sc
        return 0

    jax.lax.fori_loop(0, _NS, strip, 0, unroll=False)


_G = 4


def _nms_kernel(tb_ref, ts_ref, out_ref, s_scr, a2_scr):
    for g in range(_G):
        s_scr[g] = ts_ref[g]
        a2_scr[g] = ((tb_ref[g, 2] - tb_ref[g, 0]) *
                     (tb_ref[g, 3] - tb_ref[g, 1]))
    fi = (jax.lax.broadcasted_iota(jnp.int32, (_PR, 128), 0) * 128 +
          jax.lax.broadcasted_iota(jnp.int32, (_PR, 128), 1))
    lane = jax.lax.broadcasted_iota(jnp.int32, (1, 128), 1)
    m0 = (lane == 0).astype(jnp.float32)
    m1 = (lane == 1).astype(jnp.float32)
    m2 = (lane == 2).astype(jnp.float32)
    m3 = (lane == 3).astype(jnp.float32)
    m4 = (lane == 4).astype(jnp.float32)
    ninf = jnp.float32(-jnp.inf)

    def body(i, _):
        for g in range(_G):
            s = s_scr[g]
            m = jnp.max(s, axis=(0, 1), keepdims=True)
            ok = m > ninf
            idx = jnp.min(jnp.where(s == m, fi, jnp.int32(2 ** 30)),
                          axis=(0, 1), keepdims=True)
            sel = fi == idx
            by1 = jnp.max(jnp.where(sel, tb_ref[g, 0], ninf), axis=(0, 1), keepdims=True)
            bx1 = jnp.max(jnp.where(sel, tb_ref[g, 1], ninf), axis=(0, 1), keepdims=True)
            by2 = jnp.max(jnp.where(sel, tb_ref[g, 2], ninf), axis=(0, 1), keepdims=True)
            bx2 = jnp.max(jnp.where(sel, tb_ref[g, 3], ninf), axis=(0, 1), keepdims=True)
            yy1 = jnp.maximum(by1, tb_ref[g, 0])
            xx1 = jnp.maximum(bx1, tb_ref[g, 1])
            yy2 = jnp.minimum(by2, tb_ref[g, 2])
            xx2 = jnp.minimum(bx2, tb_ref[g, 3])
            inter = jnp.maximum(yy2 - yy1, 0.0) * jnp.maximum(xx2 - xx1, 0.0)
            a1 = (by2 - by1) * (bx2 - bx1)
            iou = inter / (a1 + a2_scr[g] - inter + 1e-9)
            s_scr[g] = jnp.where(iou <= _NMS_THRESH, s, ninf)
            okf = jnp.where(ok, 1.0, 0.0)
            row = okf * (m0 * by1 + m1 * bx1 + m2 * by2 + m3 * bx2 +
                         m4 * jnp.where(ok, m, 0.0))
            out_ref[g, pl.ds(i, 1), :] = row
        return 0

    jax.lax.fori_loop(0, _POST_NMS, body, 0, unroll=False)


def _run_heads(xpad, wcol, b1, wr, br, wc, bc, anc):
    return pl.pallas_call(
        _head_kernel,
        grid=(_B,),
        in_specs=[
            pl.BlockSpec((1, 66, 66, 512), lambda b: (b, 0, 0, 0)),
            pl.BlockSpec((4608, 512), lambda b: (0, 0)),
            pl.BlockSpec((1, 512), lambda b: (0, 0)),
            pl.BlockSpec((512, 36), lambda b: (0, 0)),
            pl.BlockSpec((1, 36), lambda b: (0, 0)),
            pl.BlockSpec((512, 18), lambda b: (0, 0)),
            pl.BlockSpec((1, 18), lambda b: (0, 0)),
            pl.BlockSpec((36, 4096), lambda b: (0, 0)),
        ],
        out_specs=[
            pl.BlockSpec((1, 36, 4096), lambda b: (b, 0, 0)),
            pl.BlockSpec((1, 9, 4096), lambda b: (b, 0, 0)),
        ],
        out_shape=[
            jax.ShapeDtypeStruct((_B, 36, 4096), jnp.float32),
            jax.ShapeDtypeStruct((_B, 9, 4096), jnp.float32),
        ],
        compiler_params=pltpu.CompilerParams(
            dimension_semantics=("parallel",),
            vmem_limit_bytes=100 * 1024 * 1024,
        ),
        name="rpn_heads",
    )(xpad, wcol, b1, wr, br, wc, bc, anc)


def _run_nms(tb_p, ts_p):
    return pl.pallas_call(
        _nms_kernel,
        grid=(_B // _G,),
        in_specs=[
            pl.BlockSpec((_G, 4, _PR, 128), lambda b: (b, 0, 0, 0)),
            pl.BlockSpec((_G, _PR, 128), lambda b: (b, 0, 0)),
        ],
        out_specs=pl.BlockSpec((_G, _POST_NMS, 128), lambda b: (b, 0, 0)),
        out_shape=jax.ShapeDtypeStruct((_B, _POST_NMS, 128), jnp.float32),
        scratch_shapes=[pltpu.VMEM((_G, _PR, 128), jnp.float32),
                        pltpu.VMEM((_G, _PR, 128), jnp.float32)],
        compiler_params=pltpu.CompilerParams(
            dimension_semantics=("parallel",),
        ),
        name="rpn_nms",
    )(tb_p, ts_p)


def kernel(feat_map, conv1_w, conv1_b, reg_w, reg_b, cls_w, cls_b, anchors):
    xpad = jnp.pad(feat_map.transpose(0, 2, 3, 1),
                   ((0, 0), (1, 1), (1, 1), (0, 0)))
    wcol = conv1_w.transpose(2, 3, 1, 0).reshape(9 * 512, 512)
    b1 = conv1_b.reshape(1, 512)
    wr = reg_w.reshape(_A * 4, 512).T
    br = reg_b.reshape(1, _A * 4)
    wc = cls_w.reshape(_A * 2, 512).T
    bc = cls_b.reshape(1, _A * 2)
    anc = anchors.reshape(4096, _A * 4).T

    boxesp, scp = _run_heads(xpad, wcol, b1, wr, br, wc, bc, anc)
    boxes = boxesp.transpose(0, 2, 1).reshape(_B, _H * _W * _A, 4)
    sc = scp.transpose(0, 2, 1).reshape(_B, _H * _W * _A)

    top_s, top_i = jax.lax.top_k(sc, _PRE_NMS)
    tb = jnp.take_along_axis(boxes, top_i[..., None], axis=1)

    pad_n = _P - _PRE_NMS
    ts_p = jnp.concatenate(
        [top_s, jnp.full((_B, pad_n), -jnp.inf, jnp.float32)], axis=1
    ).reshape(_B, _PR, 128)
    tb_p = jnp.concatenate(
        [tb, jnp.zeros((_B, pad_n, 4), jnp.float32)], axis=1
    ).transpose(0, 2, 1).reshape(_B, 4, _PR, 128)

    out = _run_nms(tb_p, ts_p)
    return out[:, :, :4], out[:, :, 4]


# speedup vs baseline: 4.2505x; 1.2739x over previous
import jax
import jax.numpy as jnp
from jax.experimental import pallas as pl
from jax.experimental.pallas import tpu as pltpu

_PRE_NMS = 6000
_POST_NMS = 300
_NMS_THRESH = 0.7
_MIN_SIZE = 16.0
_B, _C, _H, _W = 8, 512, 64, 64
_A = 9
_IMG = 1024.0
_P = 6016
_PR = _P // 128
_TH = 8
_NS = _H // _TH
_M = _TH * _W


def _head_kernel(x_ref, wcol_ref, b1_ref, wr_ref, br_ref, wc_ref, bc_ref,
                 anc_ref, boxes_ref, sc_ref):
    def strip(s, _):
        r0 = s * _TH
        acc = None
        for t in range(9):
            dy, dx = t // 3, t % 3
            xs = x_ref[0, pl.ds(r0 + dy, _TH), dx:dx + 64, :].reshape(_M, 512)
            p = jnp.dot(xs, wcol_ref[pl.ds(t * 512, 512), :],
                        preferred_element_type=jnp.float32)
            acc = p if acc is None else acc + p
        hid = jnp.maximum(acc + b1_ref[...], 0.0)
        locs = jnp.dot(hid, wr_ref[...],
                       preferred_element_type=jnp.float32) + br_ref[...]
        cls = jnp.dot(hid, wc_ref[...],
                      preferred_element_type=jnp.float32) + bc_ref[...]

        clsT = cls.transpose(1, 0)
        c0 = clsT[0:_A, :]
        c1 = clsT[_A:2 * _A, :]
        mx = jnp.maximum(c0, c1)
        e0 = jnp.exp(c0 - mx)
        e1 = jnp.exp(c1 - mx)
        fg = e1 / (e0 + e1)

        hw0 = r0 * 64
        locsT = locs.transpose(1, 0)
        l0 = locsT[0 * _A:1 * _A, :]
        l1 = locsT[1 * _A:2 * _A, :]
        l2 = locsT[2 * _A:3 * _A, :]
        l3 = locsT[3 * _A:4 * _A, :]
        ay1 = anc_ref[0 * _A:1 * _A, pl.ds(hw0, _M)]
        ax1 = anc_ref[1 * _A:2 * _A, pl.ds(hw0, _M)]
        ay2 = anc_ref[2 * _A:3 * _A, pl.ds(hw0, _M)]
        ax2 = anc_ref[3 * _A:4 * _A, pl.ds(hw0, _M)]
        ah = ay2 - ay1
        aw = ax2 - ax1
        acy = ay1 + 0.5 * ah
        acx = ax1 + 0.5 * aw
        cy = l0 * ah + acy
        cx = l1 * aw + acx
        bh = jnp.exp(l2) * ah
        bw = jnp.exp(l3) * aw
        y1 = jnp.minimum(jnp.maximum(cy - 0.5 * bh, 0.0), _IMG)
        x1 = jnp.minimum(jnp.maximum(cx - 0.5 * bw, 0.0), _IMG)
        y2 = jnp.minimum(jnp.maximum(cy + 0.5 * bh, 0.0), _IMG)
        x2 = jnp.minimum(jnp.maximum(cx + 0.5 * bw, 0.0), _IMG)
        hh = y2 - y1
        ww = x2 - x1
        sc = jnp.where((hh >= _MIN_SIZE) & (ww >= _MIN_SIZE), fg, -jnp.inf)

        box = jnp.concatenate([y1, x1, y2, x2], axis=0)
        boxes_ref[0, :, pl.ds(hw0, _M)] = box
        sc_ref[0, :, pl.ds(hw0, _M)] = sc
        return 0

    jax.lax.fori_loop(0, _NS, strip, 0, unroll=False)


_G = 4


def _nms_kernel(tb_ref, ts_ref, out_ref, s_scr, a2_scr):
    for g in range(_G):
        s_scr[g] = ts_ref[g]
        a2_scr[g] = ((tb_ref[g, 2] - tb_ref[g, 0]) *
                     (tb_ref[g, 3] - tb_ref[g, 1]))
    fi = (jax.lax.broadcasted_iota(jnp.int32, (_PR, 128), 0) * 128 +
          jax.lax.broadcasted_iota(jnp.int32, (_PR, 128), 1))
    lane = jax.lax.broadcasted_iota(jnp.int32, (1, 128), 1)
    m0 = (lane == 0).astype(jnp.float32)
    m1 = (lane == 1).astype(jnp.float32)
    m2 = (lane == 2).astype(jnp.float32)
    m3 = (lane == 3).astype(jnp.float32)
    m4 = (lane == 4).astype(jnp.float32)
    ninf = jnp.float32(-jnp.inf)

    def body(i, _):
        for g in range(_G):
            s = s_scr[g]
            m = jnp.max(s, axis=(0, 1), keepdims=True)
            ok = m > ninf
            idx = jnp.min(jnp.where(s == m, fi, jnp.int32(2 ** 30)),
                          axis=(0, 1), keepdims=True)
            sel = fi == idx
            by1 = jnp.max(jnp.where(sel, tb_ref[g, 0], ninf), axis=(0, 1), keepdims=True)
            bx1 = jnp.max(jnp.where(sel, tb_ref[g, 1], ninf), axis=(0, 1), keepdims=True)
            by2 = jnp.max(jnp.where(sel, tb_ref[g, 2], ninf), axis=(0, 1), keepdims=True)
            bx2 = jnp.max(jnp.where(sel, tb_ref[g, 3], ninf), axis=(0, 1), keepdims=True)
            yy1 = jnp.maximum(by1, tb_ref[g, 0])
            xx1 = jnp.maximum(bx1, tb_ref[g, 1])
            yy2 = jnp.minimum(by2, tb_ref[g, 2])
            xx2 = jnp.minimum(bx2, tb_ref[g, 3])
            inter = jnp.maximum(yy2 - yy1, 0.0) * jnp.maximum(xx2 - xx1, 0.0)
            a1 = (by2 - by1) * (bx2 - bx1)
            iou = inter / (a1 + a2_scr[g] - inter + 1e-9)
            s_scr[g] = jnp.where(iou <= _NMS_THRESH, s, ninf)
            okf = jnp.where(ok, 1.0, 0.0)
            row = okf * (m0 * by1 + m1 * bx1 + m2 * by2 + m3 * bx2 +
                         m4 * jnp.where(ok, m, 0.0))
            out_ref[g, pl.ds(i, 1), :] = row
        return 0

    jax.lax.fori_loop(0, _POST_NMS, body, 0, unroll=False)


def _run_heads(xpad, wcol, b1, wr, br, wc, bc, anc):
    return pl.pallas_call(
        _head_kernel,
        grid=(_B,),
        in_specs=[
            pl.BlockSpec((1, 66, 66, 512), lambda b: (b, 0, 0, 0)),
            pl.BlockSpec((4608, 512), lambda b: (0, 0)),
            pl.BlockSpec((1, 512), lambda b: (0, 0)),
            pl.BlockSpec((512, 36), lambda b: (0, 0)),
            pl.BlockSpec((1, 36), lambda b: (0, 0)),
            pl.BlockSpec((512, 18), lambda b: (0, 0)),
            pl.BlockSpec((1, 18), lambda b: (0, 0)),
            pl.BlockSpec((36, 4096), lambda b: (0, 0)),
        ],
        out_specs=[
            pl.BlockSpec((1, 36, 4096), lambda b: (b, 0, 0)),
            pl.BlockSpec((1, 9, 4096), lambda b: (b, 0, 0)),
        ],
        out_shape=[
            jax.ShapeDtypeStruct((_B, 36, 4096), jnp.float32),
            jax.ShapeDtypeStruct((_B, 9, 4096), jnp.float32),
        ],
        compiler_params=pltpu.CompilerParams(
            dimension_semantics=("parallel",),
            vmem_limit_bytes=100 * 1024 * 1024,
        ),
        name="rpn_heads",
    )(xpad, wcol, b1, wr, br, wc, bc, anc)


def _run_nms(tb_p, ts_p):
    return pl.pallas_call(
        _nms_kernel,
        grid=(_B // _G,),
        in_specs=[
            pl.BlockSpec((_G, 4, _PR, 128), lambda b: (b, 0, 0, 0)),
            pl.BlockSpec((_G, _PR, 128), lambda b: (b, 0, 0)),
        ],
        out_specs=pl.BlockSpec((_G, _POST_NMS, 128), lambda b: (b, 0, 0)),
        out_shape=jax.ShapeDtypeStruct((_B, _POST_NMS, 128), jnp.float32),
        scratch_shapes=[pltpu.VMEM((_G, _PR, 128), jnp.float32),
                        pltpu.VMEM((_G, _PR, 128), jnp.float32)],
        compiler_params=pltpu.CompilerParams(
            dimension_semantics=("parallel",),
        ),
        name="rpn_nms",
    )(tb_p, ts_p)


def kernel(feat_map, conv1_w, conv1_b, reg_w, reg_b, cls_w, cls_b, anchors):
    xpad = jnp.pad(feat_map.transpose(0, 2, 3, 1),
                   ((0, 0), (1, 1), (1, 1), (0, 0)))
    wcol = conv1_w.transpose(2, 3, 1, 0).reshape(9 * 512, 512)
    b1 = conv1_b.reshape(1, 512)
    wr = reg_w.reshape(_A, 4, 512).transpose(1, 0, 2).reshape(4 * _A, 512).T
    br = reg_b.reshape(_A, 4).transpose(1, 0).reshape(1, 4 * _A)
    wc = cls_w.reshape(_A, 2, 512).transpose(1, 0, 2).reshape(2 * _A, 512).T
    bc = cls_b.reshape(_A, 2).transpose(1, 0).reshape(1, 2 * _A)
    anc = anchors.reshape(4096, _A, 4).transpose(2, 1, 0).reshape(4 * _A, 4096)

    boxesp, scp = _run_heads(xpad, wcol, b1, wr, br, wc, bc, anc)
    boxes = boxesp.reshape(_B, 4, _A, 4096).transpose(0, 3, 2, 1).reshape(
        _B, _H * _W * _A, 4)
    sc = scp.transpose(0, 2, 1).reshape(_B, _H * _W * _A)

    top_s, top_i = jax.lax.top_k(sc, _PRE_NMS)
    tb = jnp.take_along_axis(boxes, top_i[..., None], axis=1)

    pad_n = _P - _PRE_NMS
    ts_p = jnp.concatenate(
        [top_s, jnp.full((_B, pad_n), -jnp.inf, jnp.float32)], axis=1
    ).reshape(_B, _PR, 128)
    tb_p = jnp.concatenate(
        [tb, jnp.zeros((_B, pad_n, 4), jnp.float32)], axis=1
    ).transpose(0, 2, 1).reshape(_B, 4, _PR, 128)

    out = _run_nms(tb_p, ts_p)
    return out[:, :, :4], out[:, :, 4]
